# baseline (speedup 1.0000x reference)
import functools

import jax
import jax.numpy as jnp
from jax import lax
from jax.experimental import pallas as pl
from jax.experimental.pallas import tpu as pltpu
from jax.experimental.pallas import tpu_sc as plsc

_LAMBDA = 0.1
_STEP = 2.0 / 15.0


def _quantize(v):
    c = jnp.clip(v, -_LAMBDA, _LAMBDA)
    s = v - c
    t = jnp.clip(s * 7.5 + 8.0, 0.5, 15.5)
    f = t.astype(jnp.int32).astype(jnp.float32)
    q = f * _STEP - 1.0
    q = jnp.where(jnp.abs(s) < 1e-6, 0.0, q)
    return (q - v) + v


def _tc_body(x_ref, o_ref):
    o_ref[...] = _quantize(x_ref[...])


def _tc_call(x3, grid=8):
    b, r, c = x3.shape
    block = b // grid
    return pl.pallas_call(
        _tc_body,
        grid=(grid,),
        in_specs=[pl.BlockSpec((block, r, c), lambda i: (i, 0, 0))],
        out_specs=pl.BlockSpec((block, r, c), lambda i: (i, 0, 0)),
        out_shape=jax.ShapeDtypeStruct((b, r, c), x3.dtype),
    )(x3)


_NC, _NS, _L = 2, 16, 16
_NW = _NC * _NS


_UNROLL = 4


def _make_sc_call3(shape):
    b, r, c = shape
    rows = 128
    bpw = b // _NW
    nch = bpw * (r // rows)
    npair = nch // 2
    cvec = c // _L

    @functools.partial(
        pl.kernel,
        mesh=plsc.VectorSubcoreMesh(core_axis_name="c", subcore_axis_name="s"),
        out_type=jax.ShapeDtypeStruct(shape, jnp.float32),
        scratch_types=[
            pltpu.VMEM((rows, c), jnp.float32),
            pltpu.VMEM((rows, c), jnp.float32),
            pltpu.VMEM((rows, c), jnp.float32),
            pltpu.VMEM((rows, c), jnp.float32),
            pltpu.SemaphoreType.DMA,
            pltpu.SemaphoreType.DMA,
            pltpu.SemaphoreType.DMA,
            pltpu.SemaphoreType.DMA,
        ],
    )
    def sc_quantize(x_hbm, o_hbm, in0, in1, out0, out1, si0, si1, so0, so1):
        wid = lax.axis_index("s") * _NC + lax.axis_index("c")
        ins, outs = (in0, in1), (out0, out1)
        sis, sos = (si0, si1), (so0, so1)

        def addr(ci):
            bi = wid * bpw + lax.shift_right_logical(ci, 3)
            r0 = lax.bitwise_and(ci, 7) * rows
            return bi, r0

        def in_copy(ci, bf):
            bi, r0 = addr(ci)
            return pltpu.make_async_copy(
                x_hbm.at[bi, pl.ds(r0, rows), :], ins[bf], sis[bf])

        def out_copy(ci, bf):
            bi, r0 = addr(ci)
            return pltpu.make_async_copy(
                outs[bf], o_hbm.at[bi, pl.ds(r0, rows), :], sos[bf])

        def compute(bf):
            def body(i, carry):
                for u in range(cvec):
                    v = ins[bf][i, pl.ds(u * _L, _L)]
                    outs[bf][i, pl.ds(u * _L, _L)] = _quantize(v)
                return carry

            lax.fori_loop(0, rows, body, 0)

        in_copy(0, 0).start()
        in_copy(1, 1).start()

        def pair(p, carry):
            for bf in range(2):
                ci = p * 2 + bf
                in_copy(ci, bf).wait()

                @pl.when(p > 0)
                def _():
                    out_copy(ci, bf).wait()

                compute(bf)
                out_copy(ci, bf).start()

                @pl.when(p < npair - 1)
                def _():
                    in_copy(ci + 2, bf).start()

            return carry

        lax.fori_loop(0, npair, pair, 0)
        out_copy(nch - 2, 0).wait()
        out_copy(nch - 1, 1).wait()

    return sc_quantize


def _make_sc_inplace(shape):
    b, r, c = shape
    rows = 128
    bpw = b // _NW
    nch = bpw * (r // rows)
    npair = nch // 2
    cvec = c // _L

    @functools.partial(
        pl.kernel,
        mesh=plsc.VectorSubcoreMesh(core_axis_name="c", subcore_axis_name="s"),
        scratch_types=[
            pltpu.VMEM((rows, c), jnp.float32),
            pltpu.VMEM((rows, c), jnp.float32),
            pltpu.VMEM((rows, c), jnp.float32),
            pltpu.VMEM((rows, c), jnp.float32),
            pltpu.SemaphoreType.DMA,
            pltpu.SemaphoreType.DMA,
            pltpu.SemaphoreType.DMA,
            pltpu.SemaphoreType.DMA,
        ],
    )
    def sc_quantize(x_hbm, in0, in1, out0, out1, si0, si1, so0, so1):
        wid = lax.axis_index("s") * _NC + lax.axis_index("c")
        ins, outs = (in0, in1), (out0, out1)
        sis, sos = (si0, si1), (so0, so1)

        def addr(ci):
            bi = wid * bpw + lax.shift_right_logical(ci, 3)
            r0 = lax.bitwise_and(ci, 7) * rows
            return bi, r0

        def in_copy(ci, bf):
            bi, r0 = addr(ci)
            return pltpu.make_async_copy(
                x_hbm.at[bi, pl.ds(r0, rows), :], ins[bf], sis[bf])

        def out_copy(ci, bf):
            bi, r0 = addr(ci)
            return pltpu.make_async_copy(
                outs[bf], x_hbm.at[bi, pl.ds(r0, rows), :], sos[bf])

        def compute(bf):
            def body(i, carry):
                for u in range(cvec):
                    v = ins[bf][i, pl.ds(u * _L, _L)]
                    outs[bf][i, pl.ds(u * _L, _L)] = _quantize(v)
                return carry

            lax.fori_loop(0, rows, body, 0)

        in_copy(0, 0).start()
        in_copy(1, 1).start()

        def pair(p, carry):
            for bf in range(2):
                ci = p * 2 + bf
                in_copy(ci, bf).wait()

                @pl.when(p > 0)
                def _():
                    out_copy(ci, bf).wait()

                compute(bf)
                out_copy(ci, bf).start()

                @pl.when(p < npair - 1)
                def _():
                    in_copy(ci + 2, bf).start()

            return carry

        lax.fori_loop(0, npair, pair, 0)
        out_copy(nch - 2, 0).wait()
        out_copy(nch - 1, 1).wait()

    return sc_quantize


def _make_sc_call(n):
    per_w = n // _NW
    ch = min(per_w, 16384)
    nch = per_w // ch

    @functools.partial(
        pl.kernel,
        mesh=plsc.VectorSubcoreMesh(core_axis_name="c", subcore_axis_name="s"),
        out_type=jax.ShapeDtypeStruct((n,), jnp.float32),
        scratch_types=[
            pltpu.VMEM((ch,), jnp.float32),
            pltpu.VMEM((ch,), jnp.float32),
            pltpu.VMEM((ch,), jnp.float32),
            pltpu.VMEM((ch,), jnp.float32),
            pltpu.SemaphoreType.DMA,
            pltpu.SemaphoreType.DMA,
            pltpu.SemaphoreType.DMA,
            pltpu.SemaphoreType.DMA,
        ],
    )
    def sc_quantize(x_hbm, o_hbm, in0, in1, out0, out1, si0, si1, so0, so1):
        wid = lax.axis_index("s") * _NC + lax.axis_index("c")
        base = wid * per_w
        ins, outs = (in0, in1), (out0, out1)
        sis, sos = (si0, si1), (so0, so1)

        def in_copy(ci, b):
            return pltpu.make_async_copy(
                x_hbm.at[pl.ds(base + ci * ch, ch)], ins[b], sis[b])

        def out_copy(ci, b):
            return pltpu.make_async_copy(
                outs[b], o_hbm.at[pl.ds(base + ci * ch, ch)], sos[b])

        in_copy(0, 0).start()
        if nch > 1:
            in_copy(1, 1).start()
        for ci in range(nch):
            b = ci % 2
            in_copy(ci, b).wait()
            if ci >= 2:
                out_copy(ci - 2, b).wait()

            def body(i, carry, b=b):
                j = i * _UNROLL * _L
                for u in range(_UNROLL):
                    v = ins[b][pl.ds(j + u * _L, _L)]
                    outs[b][pl.ds(j + u * _L, _L)] = _quantize(v)
                return carry

            lax.fori_loop(0, ch // (_L * _UNROLL), body, 0)
            out_copy(ci, b).start()
            if ci + 2 < nch:
                in_copy(ci + 2, b).start()
        if nch > 1:
            out_copy(nch - 2, (nch - 2) % 2).wait()
        out_copy(nch - 1, (nch - 1) % 2).wait()

    return sc_quantize


_SC_CALL_CACHE = {}


def _sc_call(xf):
    n = xf.shape[0]
    if n not in _SC_CALL_CACHE:
        _SC_CALL_CACHE[n] = _make_sc_call(n)
    return _SC_CALL_CACHE[n](xf)


_SC_BATCH_SPLIT = 32


def kernel(x):
    x_sc = x[:_SC_BATCH_SPLIT]
    x_tc = x[_SC_BATCH_SPLIT:]
    key = ("sc3", x_sc.shape)
    if key not in _SC_CALL_CACHE:
        _SC_CALL_CACHE[key] = _make_sc_call3(x_sc.shape)
    o_sc = _SC_CALL_CACHE[key](x_sc)
    o_tc = _tc_call(x_tc, grid=4)
    return jnp.concatenate([o_sc, o_tc], axis=0)


# speedup vs baseline: 1.1935x; 1.1935x over previous
import functools

import jax
import jax.numpy as jnp
from jax import lax
from jax.experimental import pallas as pl
from jax.experimental.pallas import tpu as pltpu
from jax.experimental.pallas import tpu_sc as plsc

_LAMBDA = 0.1
_STEP = 2.0 / 15.0


def _quantize(v):
    c = jnp.clip(v, -_LAMBDA, _LAMBDA)
    s = v - c
    t = jnp.clip(s * 7.5 + 8.0, 0.5, 15.5)
    f = t.astype(jnp.int32).astype(jnp.float32)
    q = f * _STEP - 1.0
    return jnp.where(jnp.abs(s) < 1e-6, 0.0, q)


def _tc_body(x_ref, o_ref):
    o_ref[...] = _quantize(x_ref[...])


def _tc_call(x3, grid=8):
    b, r, c = x3.shape
    block = b // grid
    return pl.pallas_call(
        _tc_body,
        grid=(grid,),
        in_specs=[pl.BlockSpec((block, r, c), lambda i: (i, 0, 0))],
        out_specs=pl.BlockSpec((block, r, c), lambda i: (i, 0, 0)),
        out_shape=jax.ShapeDtypeStruct((b, r, c), x3.dtype),
    )(x3)


_NC, _NS, _L = 2, 16, 16
_NW = _NC * _NS


_UNROLL = 4


def _make_sc_call3(shape):
    b, r, c = shape
    rows = 128
    bpw = b // _NW
    nch = bpw * (r // rows)
    npair = nch // 2
    cvec = c // _L

    @functools.partial(
        pl.kernel,
        mesh=plsc.VectorSubcoreMesh(core_axis_name="c", subcore_axis_name="s"),
        out_type=jax.ShapeDtypeStruct(shape, jnp.float32),
        scratch_types=[
            pltpu.VMEM((rows, c), jnp.float32),
            pltpu.VMEM((rows, c), jnp.float32),
            pltpu.VMEM((rows, c), jnp.float32),
            pltpu.VMEM((rows, c), jnp.float32),
            pltpu.SemaphoreType.DMA,
            pltpu.SemaphoreType.DMA,
            pltpu.SemaphoreType.DMA,
            pltpu.SemaphoreType.DMA,
        ],
    )
    def sc_quantize(x_hbm, o_hbm, in0, in1, out0, out1, si0, si1, so0, so1):
        wid = lax.axis_index("s") * _NC + lax.axis_index("c")
        ins, outs = (in0, in1), (out0, out1)
        sis, sos = (si0, si1), (so0, so1)

        def addr(ci):
            bi = wid * bpw + lax.shift_right_logical(ci, 3)
            r0 = lax.bitwise_and(ci, 7) * rows
            return bi, r0

        def in_copy(ci, bf):
            bi, r0 = addr(ci)
            return pltpu.make_async_copy(
                x_hbm.at[bi, pl.ds(r0, rows), :], ins[bf], sis[bf])

        def out_copy(ci, bf):
            bi, r0 = addr(ci)
            return pltpu.make_async_copy(
                outs[bf], o_hbm.at[bi, pl.ds(r0, rows), :], sos[bf])

        def compute(bf):
            def body(i, carry):
                for k in range(2):
                    for u in range(cvec):
                        v = ins[bf][i * 2 + k, pl.ds(u * _L, _L)]
                        outs[bf][i * 2 + k, pl.ds(u * _L, _L)] = _quantize(v)
                return carry

            lax.fori_loop(0, rows // 2, body, 0)

        in_copy(0, 0).start()
        in_copy(1, 1).start()

        def pair(p, carry):
            for bf in range(2):
                ci = p * 2 + bf
                in_copy(ci, bf).wait()

                @pl.when(p > 0)
                def _():
                    out_copy(ci, bf).wait()

                compute(bf)
                out_copy(ci, bf).start()

                @pl.when(p < npair - 1)
                def _():
                    in_copy(ci + 2, bf).start()

            return carry

        lax.fori_loop(0, npair, pair, 0)
        out_copy(nch - 2, 0).wait()
        out_copy(nch - 1, 1).wait()

    return sc_quantize


def _make_sc_inplace(shape):
    b, r, c = shape
    rows = 128
    bpw = b // _NW
    nch = bpw * (r // rows)
    npair = nch // 2
    cvec = c // _L

    @functools.partial(
        pl.kernel,
        mesh=plsc.VectorSubcoreMesh(core_axis_name="c", subcore_axis_name="s"),
        scratch_types=[
            pltpu.VMEM((rows, c), jnp.float32),
            pltpu.VMEM((rows, c), jnp.float32),
            pltpu.VMEM((rows, c), jnp.float32),
            pltpu.VMEM((rows, c), jnp.float32),
            pltpu.SemaphoreType.DMA,
            pltpu.SemaphoreType.DMA,
            pltpu.SemaphoreType.DMA,
            pltpu.SemaphoreType.DMA,
        ],
    )
    def sc_quantize(x_hbm, in0, in1, out0, out1, si0, si1, so0, so1):
        wid = lax.axis_index("s") * _NC + lax.axis_index("c")
        ins, outs = (in0, in1), (out0, out1)
        sis, sos = (si0, si1), (so0, so1)

        def addr(ci):
            bi = wid * bpw + lax.shift_right_logical(ci, 3)
            r0 = lax.bitwise_and(ci, 7) * rows
            return bi, r0

        def in_copy(ci, bf):
            bi, r0 = addr(ci)
            return pltpu.make_async_copy(
                x_hbm.at[bi, pl.ds(r0, rows), :], ins[bf], sis[bf])

        def out_copy(ci, bf):
            bi, r0 = addr(ci)
            return pltpu.make_async_copy(
                outs[bf], x_hbm.at[bi, pl.ds(r0, rows), :], sos[bf])

        def compute(bf):
            def body(i, carry):
                for k in range(2):
                    for u in range(cvec):
                        v = ins[bf][i * 2 + k, pl.ds(u * _L, _L)]
                        outs[bf][i * 2 + k, pl.ds(u * _L, _L)] = _quantize(v)
                return carry

            lax.fori_loop(0, rows // 2, body, 0)

        in_copy(0, 0).start()
        in_copy(1, 1).start()

        def pair(p, carry):
            for bf in range(2):
                ci = p * 2 + bf
                in_copy(ci, bf).wait()

                @pl.when(p > 0)
                def _():
                    out_copy(ci, bf).wait()

                compute(bf)
                out_copy(ci, bf).start()

                @pl.when(p < npair - 1)
                def _():
                    in_copy(ci + 2, bf).start()

            return carry

        lax.fori_loop(0, npair, pair, 0)
        out_copy(nch - 2, 0).wait()
        out_copy(nch - 1, 1).wait()

    return sc_quantize


def _make_sc_call(n):
    per_w = n // _NW
    ch = min(per_w, 16384)
    nch = per_w // ch

    @functools.partial(
        pl.kernel,
        mesh=plsc.VectorSubcoreMesh(core_axis_name="c", subcore_axis_name="s"),
        out_type=jax.ShapeDtypeStruct((n,), jnp.float32),
        scratch_types=[
            pltpu.VMEM((ch,), jnp.float32),
            pltpu.VMEM((ch,), jnp.float32),
            pltpu.VMEM((ch,), jnp.float32),
            pltpu.VMEM((ch,), jnp.float32),
            pltpu.SemaphoreType.DMA,
            pltpu.SemaphoreType.DMA,
            pltpu.SemaphoreType.DMA,
            pltpu.SemaphoreType.DMA,
        ],
    )
    def sc_quantize(x_hbm, o_hbm, in0, in1, out0, out1, si0, si1, so0, so1):
        wid = lax.axis_index("s") * _NC + lax.axis_index("c")
        base = wid * per_w
        ins, outs = (in0, in1), (out0, out1)
        sis, sos = (si0, si1), (so0, so1)

        def in_copy(ci, b):
            return pltpu.make_async_copy(
                x_hbm.at[pl.ds(base + ci * ch, ch)], ins[b], sis[b])

        def out_copy(ci, b):
            return pltpu.make_async_copy(
                outs[b], o_hbm.at[pl.ds(base + ci * ch, ch)], sos[b])

        in_copy(0, 0).start()
        if nch > 1:
            in_copy(1, 1).start()
        for ci in range(nch):
            b = ci % 2
            in_copy(ci, b).wait()
            if ci >= 2:
                out_copy(ci - 2, b).wait()

            def body(i, carry, b=b):
                j = i * _UNROLL * _L
                for u in range(_UNROLL):
                    v = ins[b][pl.ds(j + u * _L, _L)]
                    outs[b][pl.ds(j + u * _L, _L)] = _quantize(v)
                return carry

            lax.fori_loop(0, ch // (_L * _UNROLL), body, 0)
            out_copy(ci, b).start()
            if ci + 2 < nch:
                in_copy(ci + 2, b).start()
        if nch > 1:
            out_copy(nch - 2, (nch - 2) % 2).wait()
        out_copy(nch - 1, (nch - 1) % 2).wait()

    return sc_quantize


_SC_CALL_CACHE = {}


def _sc_call(xf):
    n = xf.shape[0]
    if n not in _SC_CALL_CACHE:
        _SC_CALL_CACHE[n] = _make_sc_call(n)
    return _SC_CALL_CACHE[n](xf)


def kernel(x):
    key = ("sc3", x.shape)
    if key not in _SC_CALL_CACHE:
        _SC_CALL_CACHE[key] = _make_sc_call3(x.shape)
    return _SC_CALL_CACHE[key](x)


# speedup vs baseline: 1.1951x; 1.0014x over previous
import functools

import jax
import jax.numpy as jnp
from jax import lax
from jax.experimental import pallas as pl
from jax.experimental.pallas import tpu as pltpu
from jax.experimental.pallas import tpu_sc as plsc

_LAMBDA = 0.1
_STEP = 2.0 / 15.0

_NC, _NS, _L = 2, 16, 16
_NW = _NC * _NS


def _quantize(v):
    c = jnp.clip(v, -_LAMBDA, _LAMBDA)
    s = v - c
    t = jnp.clip(s * 7.5 + 8.0, 0.5, 15.5)
    f = t.astype(jnp.int32).astype(jnp.float32)
    q = f * _STEP - 1.0
    return jnp.where(jnp.abs(s) < 1e-6, 0.0, q)


def _make_sc_call(shape):
    b, r, c = shape
    rows = 128
    bpw = b // _NW
    cpb = r // rows
    nch = bpw * cpb
    npair = nch // 2
    cvec = c // _L

    @functools.partial(
        pl.kernel,
        mesh=plsc.VectorSubcoreMesh(core_axis_name="c", subcore_axis_name="s"),
        out_type=jax.ShapeDtypeStruct(shape, jnp.float32),
        scratch_types=[
            pltpu.VMEM((rows, c), jnp.float32),
            pltpu.VMEM((rows, c), jnp.float32),
            pltpu.VMEM((rows, c), jnp.float32),
            pltpu.VMEM((rows, c), jnp.float32),
            pltpu.SemaphoreType.DMA,
            pltpu.SemaphoreType.DMA,
            pltpu.SemaphoreType.DMA,
            pltpu.SemaphoreType.DMA,
        ],
    )
    def sc_quantize(x_hbm, o_hbm, in0, in1, out0, out1, si0, si1, so0, so1):
        wid = lax.axis_index("s") * _NC + lax.axis_index("c")
        ins, outs = (in0, in1), (out0, out1)
        sis, sos = (si0, si1), (so0, so1)

        def addr(ci):
            bi = wid * bpw + lax.shift_right_logical(ci, 3)
            r0 = lax.bitwise_and(ci, 7) * rows
            return bi, r0

        def in_copy(ci, bf):
            bi, r0 = addr(ci)
            return pltpu.make_async_copy(
                x_hbm.at[bi, pl.ds(r0, rows), :], ins[bf], sis[bf])

        def out_copy(ci, bf):
            bi, r0 = addr(ci)
            return pltpu.make_async_copy(
                outs[bf], o_hbm.at[bi, pl.ds(r0, rows), :], sos[bf])

        def compute(bf):
            def body(i, carry):
                for k in range(2):
                    for u in range(cvec):
                        v = ins[bf][i * 2 + k, pl.ds(u * _L, _L)]
                        outs[bf][i * 2 + k, pl.ds(u * _L, _L)] = _quantize(v)
                return carry

            lax.fori_loop(0, rows // 2, body, 0)

        in_copy(0, 0).start()
        in_copy(1, 1).start()

        def pair(p, carry):
            for bf in range(2):
                ci = p * 2 + bf
                in_copy(ci, bf).wait()

                @pl.when(p > 0)
                def _():
                    out_copy(ci, bf).wait()

                compute(bf)
                out_copy(ci, bf).start()

                @pl.when(p < npair - 1)
                def _():
                    in_copy(ci + 2, bf).start()

            return carry

        lax.fori_loop(0, npair, pair, 0)
        out_copy(nch - 2, 0).wait()
        out_copy(nch - 1, 1).wait()

    return sc_quantize


_SC_CALL_CACHE = {}


def kernel(x):
    if x.shape not in _SC_CALL_CACHE:
        _SC_CALL_CACHE[x.shape] = _make_sc_call(x.shape)
    return _SC_CALL_CACHE[x.shape](x)
